# Initial kernel scaffold; baseline (speedup 1.0000x reference)
import jax
import jax.numpy as jnp
from jax.experimental import pallas as pl


def kernel(x, edge_index, n_nodes, W1, b1, W2, b2, W3, b3):
    raise NotImplementedError("write your pallas kernel here")


# speedup vs baseline: 4.7434x; 4.7434x over previous
import functools

import jax
import jax.numpy as jnp
from jax import lax
from jax.experimental import pallas as pl
from jax.experimental.pallas import tpu as pltpu
from jax.experimental.pallas import tpu_sc as plsc

N = 10000
NPAD = 10240
E = 320000
D1 = 128
D2 = 64
NC = 2
NS = 16
K = 80
DEGW = 8
ZR = 64
RPT = NPAD // NS
EPT = E // (NC * NS)
NB = EPT // K

_sc_mesh = plsc.VectorSubcoreMesh(core_axis_name="c", subcore_axis_name="s")


def _agg1_body(y, row, col, zeros, ones, zeros8, out, degout,
               ridx, cidx, rows, zv, z8v, onesv, sem, acc, dacc):
    c = lax.axis_index("c")
    s = lax.axis_index("s")
    pltpu.sync_copy(zeros, zv)
    pltpu.sync_copy(zeros8, z8v)
    pltpu.sync_copy(ones, onesv)
    base = s * RPT
    for j in range(RPT // ZR):
        pltpu.sync_copy(zv, acc.at[pl.ds(base + j * ZR, ZR)])
        pltpu.sync_copy(z8v, dacc.at[pl.ds(base + j * ZR, ZR)])
    plsc.subcore_barrier()

    ebase = (c * NS + s) * EPT

    def step(i, carry):
        off = ebase + i * K
        pltpu.sync_copy(row.at[pl.ds(off, K)], ridx)
        pltpu.sync_copy(col.at[pl.ds(off, K)], cidx)
        pltpu.async_copy(y.at[ridx], rows, sem).wait()
        pltpu.sync_copy(rows, acc.at[cidx], add=True)
        pltpu.sync_copy(onesv, dacc.at[cidx], add=True)
        return carry

    lax.fori_loop(0, NB, step, 0)
    plsc.subcore_barrier()
    pltpu.sync_copy(acc.at[pl.ds(base, RPT)], out.at[c].at[pl.ds(base, RPT)])
    pltpu.sync_copy(dacc.at[pl.ds(base, RPT)],
                    degout.at[c].at[pl.ds(base, RPT)])


_agg1 = pl.kernel(
    _agg1_body,
    out_type=[
        jax.ShapeDtypeStruct((NC, NPAD, D1), jnp.float32),
        jax.ShapeDtypeStruct((NC, NPAD, DEGW), jnp.float32),
    ],
    scratch_types=[
        pltpu.VMEM((K,), jnp.int32),
        pltpu.VMEM((K,), jnp.int32),
        pltpu.VMEM((K, D1), jnp.float32),
        pltpu.VMEM((ZR, D1), jnp.float32),
        pltpu.VMEM((ZR, DEGW), jnp.float32),
        pltpu.VMEM((K, DEGW), jnp.float32),
        pltpu.SemaphoreType.DMA,
        pltpu.VMEM_SHARED((NPAD, D1), jnp.float32),
        pltpu.VMEM_SHARED((NPAD, DEGW), jnp.float32),
    ],
    mesh=_sc_mesh,
    compiler_params=pltpu.CompilerParams(use_tc_tiling_on_sc=False),
)


def _agg2_body(y, row, col, zeros, out, ridx, cidx, rows, zv, sem, acc):
    c = lax.axis_index("c")
    s = lax.axis_index("s")
    pltpu.sync_copy(zeros, zv)
    base = s * RPT
    for j in range(RPT // ZR):
        pltpu.sync_copy(zv, acc.at[pl.ds(base + j * ZR, ZR)])
    plsc.subcore_barrier()

    ebase = (c * NS + s) * EPT

    def step(i, carry):
        off = ebase + i * K
        pltpu.sync_copy(row.at[pl.ds(off, K)], ridx)
        pltpu.sync_copy(col.at[pl.ds(off, K)], cidx)
        pltpu.async_copy(y.at[ridx], rows, sem).wait()
        pltpu.sync_copy(rows, acc.at[cidx], add=True)
        return carry

    lax.fori_loop(0, NB, step, 0)
    plsc.subcore_barrier()
    pltpu.sync_copy(acc.at[pl.ds(base, RPT)], out.at[c].at[pl.ds(base, RPT)])


_agg2 = pl.kernel(
    _agg2_body,
    out_type=jax.ShapeDtypeStruct((NC, NPAD, D2), jnp.float32),
    scratch_types=[
        pltpu.VMEM((K,), jnp.int32),
        pltpu.VMEM((K,), jnp.int32),
        pltpu.VMEM((K, D2), jnp.float32),
        pltpu.VMEM((ZR, D2), jnp.float32),
        pltpu.SemaphoreType.DMA,
        pltpu.VMEM_SHARED((NPAD, D2), jnp.float32),
    ],
    mesh=_sc_mesh,
    compiler_params=pltpu.CompilerParams(use_tc_tiling_on_sc=False),
)


def _mm_body(x_ref, w_ref, o_ref):
    o_ref[...] = jnp.dot(x_ref[...], w_ref[...],
                         preferred_element_type=jnp.float32)


def _layer_body(p_ref, d_ref, b_ref, w_ref, o_ref):
    p = p_ref[0] + p_ref[1]
    d = jnp.maximum(d_ref[0, :, 0:1] + d_ref[1, :, 0:1], 1.0)
    h = jnp.maximum(p / d + b_ref[...], 0.0)
    o_ref[...] = jnp.dot(h, w_ref[...], preferred_element_type=jnp.float32)


def _out_body(p_ref, d_ref, b2_ref, w_ref, b3_ref, o_ref):
    p = p_ref[0] + p_ref[1]
    d = jnp.maximum(d_ref[0, :, 0:1] + d_ref[1, :, 0:1], 1.0)
    h = jnp.maximum(p / d + b2_ref[...], 0.0)
    o_ref[...] = (jnp.dot(h, w_ref[...], preferred_element_type=jnp.float32)
                  + b3_ref[...])


def _mm(x, w):
    n, din = x.shape
    dout = w.shape[1]
    blk = n // 10
    return pl.pallas_call(
        _mm_body,
        grid=(10,),
        in_specs=[
            pl.BlockSpec((blk, din), lambda i: (i, 0)),
            pl.BlockSpec((din, dout), lambda i: (0, 0)),
        ],
        out_specs=pl.BlockSpec((blk, dout), lambda i: (i, 0)),
        out_shape=jax.ShapeDtypeStruct((n, dout), jnp.float32),
    )(x, w)


def _layer(part, degp, b, w):
    npad, din = part.shape[1], part.shape[2]
    dout = w.shape[1]
    blk = npad // 10
    return pl.pallas_call(
        _layer_body,
        grid=(10,),
        in_specs=[
            pl.BlockSpec((NC, blk, din), lambda i: (0, i, 0)),
            pl.BlockSpec((NC, blk, DEGW), lambda i: (0, i, 0)),
            pl.BlockSpec((1, din), lambda i: (0, 0)),
            pl.BlockSpec((din, dout), lambda i: (0, 0)),
        ],
        out_specs=pl.BlockSpec((blk, dout), lambda i: (i, 0)),
        out_shape=jax.ShapeDtypeStruct((npad, dout), jnp.float32),
    )(part, degp, b, w)


def _out_layer(part, degp, b2, w, b3):
    din = part.shape[2]
    dout = w.shape[1]
    blk = N // 10
    return pl.pallas_call(
        _out_body,
        grid=(10,),
        in_specs=[
            pl.BlockSpec((NC, blk, din), lambda i: (0, i, 0)),
            pl.BlockSpec((NC, blk, DEGW), lambda i: (0, i, 0)),
            pl.BlockSpec((1, din), lambda i: (0, 0)),
            pl.BlockSpec((din, dout), lambda i: (0, 0)),
            pl.BlockSpec((1, dout), lambda i: (0, 0)),
        ],
        out_specs=pl.BlockSpec((blk, dout), lambda i: (i, 0)),
        out_shape=jax.ShapeDtypeStruct((N, dout), jnp.float32),
    )(part, degp, b2, w, b3)


def kernel(x, edge_index, n_nodes, W1, b1, W2, b2, W3, b3):
    row = edge_index[0]
    col = edge_index[1]
    zeros1 = jnp.zeros((ZR, D1), jnp.float32)
    zeros2 = jnp.zeros((ZR, D2), jnp.float32)
    zeros8 = jnp.zeros((ZR, DEGW), jnp.float32)
    ones8 = jnp.ones((K, DEGW), jnp.float32)

    y1 = _mm(x, W1)
    part1, degp = _agg1(y1, row, col, zeros1, ones8, zeros8)
    y2 = _layer(part1, degp, b1.reshape(1, D1), W2)
    part2 = _agg2(y2, row, col, zeros2)
    out = _out_layer(part2, degp, b2.reshape(1, D2), W3, b3.reshape(1, D2))
    return out


# speedup vs baseline: 7.4612x; 1.5730x over previous
import functools

import jax
import jax.numpy as jnp
from jax import lax
from jax.experimental import pallas as pl
from jax.experimental.pallas import tpu as pltpu
from jax.experimental.pallas import tpu_sc as plsc

N = 10000
NPAD = 10240
E = 320000
D1 = 128
D2 = 64
NC = 2
NS = 16
NT = NC * NS
K = 100
NB = 100
NBUF = 2
DEGW = 8
RPT = NPAD // NS

_sc_mesh = plsc.VectorSubcoreMesh(core_axis_name="c", subcore_axis_name="s")


def _make_agg(D):
    def body(y, row2d, col2d, zeros, out, ridx, cidx, acc, bufs,
             gsems, ssems):
        c = lax.axis_index("c")
        s = lax.axis_index("s")
        tid = c * NS + s

        pltpu.sync_copy(row2d.at[tid], ridx)
        pltpu.sync_copy(col2d.at[tid], cidx)
        base = s * RPT
        pltpu.sync_copy(zeros, acc.at[pl.ds(base, RPT)])
        plsc.subcore_barrier()

        def pstep(b, carry):
            pltpu.make_async_copy(
                y.at[ridx.at[b]], bufs.at[b], gsems.at[b]).start()
            return carry

        lax.fori_loop(0, NBUF, pstep, 0)

        def step(i, carry):
            @pl.when(i < NB)
            def _work():
                b = lax.rem(i, NBUF)
                pltpu.make_async_copy(y.at[ridx.at[i]], bufs.at[b],
                                      gsems.at[b]).wait()
                pltpu.make_async_copy(bufs.at[b], acc.at[cidx.at[i]],
                                      ssems.at[b]).start(add=True)

            @pl.when(i >= 1)
            def _retire():
                pi = i - 1
                pb = lax.rem(pi, NBUF)
                pltpu.make_async_copy(bufs.at[pb], acc.at[cidx.at[pi]],
                                      ssems.at[pb]).wait()

                @pl.when(pi + NBUF < NB)
                def _refill():
                    pltpu.make_async_copy(y.at[ridx.at[pi + NBUF]],
                                          bufs.at[pb], gsems.at[pb]).start()

            return carry

        lax.fori_loop(0, NB + 1, step, 0)

        plsc.subcore_barrier()
        pltpu.sync_copy(acc.at[pl.ds(base, RPT)],
                        out.at[c].at[pl.ds(base, RPT)])

    return pl.kernel(
        body,
        out_type=jax.ShapeDtypeStruct((NC, NPAD, D), jnp.float32),
        scratch_types=[
            pltpu.VMEM((NB, K), jnp.int32),
            pltpu.VMEM((NB, K), jnp.int32),
            pltpu.VMEM_SHARED((NPAD, D), jnp.float32),
            pltpu.VMEM((NBUF, K, D), jnp.float32),
            pltpu.SemaphoreType.DMA((NBUF,)),
            pltpu.SemaphoreType.DMA((NBUF,)),
        ],
        mesh=_sc_mesh,
        compiler_params=pltpu.CompilerParams(use_tc_tiling_on_sc=False),
    )


def _deg_body(col2d, ones, zeros8, degout, cidx, onesv, dacc, dsem):
    c = lax.axis_index("c")
    s = lax.axis_index("s")
    tid = c * NS + s
    pltpu.sync_copy(col2d.at[tid], cidx)
    pltpu.sync_copy(ones, onesv)
    base = s * RPT
    pltpu.sync_copy(zeros8, dacc.at[pl.ds(base, RPT)])
    plsc.subcore_barrier()

    def step(i, carry):
        pltpu.make_async_copy(onesv, dacc.at[cidx.at[i]],
                              dsem).start(add=True)

        @pl.when(i >= DEG_DEPTH)
        def _drain():
            pltpu.make_async_copy(onesv, dacc.at[cidx.at[i - DEG_DEPTH]],
                                  dsem).wait()

        return carry

    lax.fori_loop(0, NB, step, 0)

    def dstep(j, carry):
        pltpu.make_async_copy(onesv, dacc.at[cidx.at[j]], dsem).wait()
        return carry

    lax.fori_loop(0, DEG_DEPTH, dstep, 0)

    plsc.subcore_barrier()
    pltpu.sync_copy(dacc.at[pl.ds(base, RPT)],
                    degout.at[c].at[pl.ds(base, RPT)])


DEG_DEPTH = 8

_deg = pl.kernel(
    _deg_body,
    out_type=jax.ShapeDtypeStruct((NC, NPAD, DEGW), jnp.float32),
    scratch_types=[
        pltpu.VMEM((NB, K), jnp.int32),
        pltpu.VMEM((K, DEGW), jnp.float32),
        pltpu.VMEM_SHARED((NPAD, DEGW), jnp.float32),
        pltpu.SemaphoreType.DMA,
    ],
    mesh=_sc_mesh,
    compiler_params=pltpu.CompilerParams(use_tc_tiling_on_sc=False),
)


_agg64 = _make_agg(D2)


def _layer_body(pa_ref, pb_ref, d_ref, b_ref, w1a_ref, w1b_ref, w2_ref,
                o_ref):
    d = jnp.maximum(d_ref[0, :, 0:1] + d_ref[1, :, 0:1], 1.0)
    ha = (pa_ref[0] + pa_ref[1]) / d
    hb = (pb_ref[0] + pb_ref[1]) / d
    h = jnp.maximum(
        jnp.dot(ha, w1a_ref[...], preferred_element_type=jnp.float32)
        + jnp.dot(hb, w1b_ref[...], preferred_element_type=jnp.float32)
        + b_ref[...], 0.0)
    o_ref[...] = jnp.dot(h, w2_ref[...], preferred_element_type=jnp.float32)


def _out_body(p_ref, d_ref, b2_ref, w_ref, b3_ref, o_ref):
    p = p_ref[0] + p_ref[1]
    d = jnp.maximum(d_ref[0, :, 0:1] + d_ref[1, :, 0:1], 1.0)
    h = jnp.maximum(p / d + b2_ref[...], 0.0)
    o_ref[...] = (jnp.dot(h, w_ref[...], preferred_element_type=jnp.float32)
                  + b3_ref[...])


def _layer(pa, pb, degp, b1, w1a, w1b, w2):
    blk = NPAD // 10
    return pl.pallas_call(
        _layer_body,
        grid=(10,),
        in_specs=[
            pl.BlockSpec((NC, blk, D2), lambda i: (0, i, 0)),
            pl.BlockSpec((NC, blk, D2), lambda i: (0, i, 0)),
            pl.BlockSpec((NC, blk, DEGW), lambda i: (0, i, 0)),
            pl.BlockSpec((1, D1), lambda i: (0, 0)),
            pl.BlockSpec((D2, D1), lambda i: (0, 0)),
            pl.BlockSpec((D2, D1), lambda i: (0, 0)),
            pl.BlockSpec((D1, D2), lambda i: (0, 0)),
        ],
        out_specs=pl.BlockSpec((blk, D2), lambda i: (i, 0)),
        out_shape=jax.ShapeDtypeStruct((NPAD, D2), jnp.float32),
    )(pa, pb, degp, b1, w1a, w1b, w2)


def _out_layer(part, degp, b2, w, b3):
    din = part.shape[2]
    dout = w.shape[1]
    blk = N // 10
    return pl.pallas_call(
        _out_body,
        grid=(10,),
        in_specs=[
            pl.BlockSpec((NC, blk, din), lambda i: (0, i, 0)),
            pl.BlockSpec((NC, blk, DEGW), lambda i: (0, i, 0)),
            pl.BlockSpec((1, din), lambda i: (0, 0)),
            pl.BlockSpec((din, dout), lambda i: (0, 0)),
            pl.BlockSpec((1, dout), lambda i: (0, 0)),
        ],
        out_specs=pl.BlockSpec((blk, dout), lambda i: (i, 0)),
        out_shape=jax.ShapeDtypeStruct((N, dout), jnp.float32),
    )(part, degp, b2, w, b3)


def kernel(x, edge_index, n_nodes, W1, b1, W2, b2, W3, b3):
    row2d = edge_index[0].reshape(NT, NB, K)
    col2d = edge_index[1].reshape(NT, NB, K)
    zeros2 = jnp.zeros((RPT, D2), jnp.float32)
    zeros8 = jnp.zeros((RPT, DEGW), jnp.float32)
    ones8 = jnp.ones((K, DEGW), jnp.float32)
    xa = x[:, :D2]
    xb = x[:, D2:]

    degp = _deg(col2d, ones8, zeros8)
    pa = _agg64(xa, row2d, col2d, zeros2)
    pb = _agg64(xb, row2d, col2d, zeros2)
    y2 = _layer(pa, pb, degp, b1.reshape(1, D1),
                W1[:D2, :], W1[D2:, :], W2)
    part2 = _agg64(y2, row2d, col2d, zeros2)
    out = _out_layer(part2, degp, b2.reshape(1, D2), W3, b3.reshape(1, D2))
    return out


# speedup vs baseline: 10.6777x; 1.4311x over previous
import functools

import jax
import jax.numpy as jnp
from jax import lax
from jax.experimental import pallas as pl
from jax.experimental.pallas import tpu as pltpu
from jax.experimental.pallas import tpu_sc as plsc

N = 10000
NPAD = 10240
E = 320000
D1 = 128
D2 = 64
NC = 2
NS = 16
NT = NC * NS
K = 100
NB = 100
NBUF = 4
DEGW = 8
RPT = NPAD // NS

_sc_mesh = plsc.VectorSubcoreMesh(core_axis_name="c", subcore_axis_name="s")


def _make_agg(D):
    def body(y, row2d, col2d, zeros, out, ridx, cidx, acc, bufs,
             gsems, ssems):
        c = lax.axis_index("c")
        s = lax.axis_index("s")
        tid = c * NS + s

        pltpu.sync_copy(row2d.at[tid], ridx)
        pltpu.sync_copy(col2d.at[tid], cidx)
        base = s * RPT
        pltpu.sync_copy(zeros, acc.at[pl.ds(base, RPT)])
        plsc.subcore_barrier()

        def pstep(b, carry):
            pltpu.make_async_copy(
                y.at[ridx.at[b]], bufs.at[b], gsems.at[b]).start()
            return carry

        lax.fori_loop(0, NBUF, pstep, 0)

        def step(i, carry):
            @pl.when(i < NB)
            def _work():
                b = lax.rem(i, NBUF)
                pltpu.make_async_copy(y.at[ridx.at[i]], bufs.at[b],
                                      gsems.at[b]).wait()
                pltpu.make_async_copy(bufs.at[b], acc.at[cidx.at[i]],
                                      ssems.at[b]).start(add=True)

            @pl.when(i >= 1)
            def _retire():
                pi = i - 1
                pb = lax.rem(pi, NBUF)
                pltpu.make_async_copy(bufs.at[pb], acc.at[cidx.at[pi]],
                                      ssems.at[pb]).wait()

                @pl.when(pi + NBUF < NB)
                def _refill():
                    pltpu.make_async_copy(y.at[ridx.at[pi + NBUF]],
                                          bufs.at[pb], gsems.at[pb]).start()

            return carry

        lax.fori_loop(0, NB + 1, step, 0)

        plsc.subcore_barrier()
        pltpu.sync_copy(acc.at[pl.ds(base, RPT)],
                        out.at[c].at[pl.ds(base, RPT)])

    return pl.kernel(
        body,
        out_type=jax.ShapeDtypeStruct((NC, NPAD, D), jnp.float32),
        scratch_types=[
            pltpu.VMEM((NB, K), jnp.int32),
            pltpu.VMEM((NB, K), jnp.int32),
            pltpu.VMEM_SHARED((NPAD, D), jnp.float32),
            pltpu.VMEM((NBUF, K, D), jnp.float32),
            pltpu.SemaphoreType.DMA((NBUF,)),
            pltpu.SemaphoreType.DMA((NBUF,)),
        ],
        mesh=_sc_mesh,
        compiler_params=pltpu.CompilerParams(use_tc_tiling_on_sc=False),
    )


def _deg_body(col2d, ones, zeros8, degout, cidx, onesv, dacc, dsem):
    c = lax.axis_index("c")
    s = lax.axis_index("s")
    tid = c * NS + s
    pltpu.sync_copy(col2d.at[tid], cidx)
    pltpu.sync_copy(ones, onesv)
    base = s * RPT
    pltpu.sync_copy(zeros8, dacc.at[pl.ds(base, RPT)])
    plsc.subcore_barrier()

    def step(i, carry):
        pltpu.make_async_copy(onesv, dacc.at[cidx.at[i]],
                              dsem).start(add=True)

        @pl.when(i >= DEG_DEPTH)
        def _drain():
            pltpu.make_async_copy(onesv, dacc.at[cidx.at[i - DEG_DEPTH]],
                                  dsem).wait()

        return carry

    lax.fori_loop(0, NB, step, 0)

    def dstep(j, carry):
        pltpu.make_async_copy(onesv, dacc.at[cidx.at[j]], dsem).wait()
        return carry

    lax.fori_loop(0, DEG_DEPTH, dstep, 0)

    plsc.subcore_barrier()
    pltpu.sync_copy(dacc.at[pl.ds(base, RPT)],
                    degout.at[c].at[pl.ds(base, RPT)])


DEG_DEPTH = 8

_deg = pl.kernel(
    _deg_body,
    out_type=jax.ShapeDtypeStruct((NC, NPAD, DEGW), jnp.float32),
    scratch_types=[
        pltpu.VMEM((NB, K), jnp.int32),
        pltpu.VMEM((K, DEGW), jnp.float32),
        pltpu.VMEM_SHARED((NPAD, DEGW), jnp.float32),
        pltpu.SemaphoreType.DMA,
    ],
    mesh=_sc_mesh,
    compiler_params=pltpu.CompilerParams(use_tc_tiling_on_sc=False),
)


_agg64 = _make_agg(D2)


def _layer_body(pa_ref, pb_ref, d_ref, b_ref, w1a_ref, w1b_ref, w2_ref,
                o_ref):
    d = jnp.maximum(d_ref[0, :, 0:1] + d_ref[1, :, 0:1], 1.0)
    ha = (pa_ref[0] + pa_ref[1]) / d
    hb = (pb_ref[0] + pb_ref[1]) / d
    h = jnp.maximum(
        jnp.dot(ha, w1a_ref[...], preferred_element_type=jnp.float32)
        + jnp.dot(hb, w1b_ref[...], preferred_element_type=jnp.float32)
        + b_ref[...], 0.0)
    o_ref[...] = jnp.dot(h, w2_ref[...], preferred_element_type=jnp.float32)


def _out_body(p_ref, d_ref, b2_ref, w_ref, b3_ref, o_ref):
    p = p_ref[0] + p_ref[1]
    d = jnp.maximum(d_ref[0, :, 0:1] + d_ref[1, :, 0:1], 1.0)
    h = jnp.maximum(p / d + b2_ref[...], 0.0)
    o_ref[...] = (jnp.dot(h, w_ref[...], preferred_element_type=jnp.float32)
                  + b3_ref[...])


def _layer(pa, pb, degp, b1, w1a, w1b, w2):
    blk = NPAD // 10
    return pl.pallas_call(
        _layer_body,
        grid=(10,),
        in_specs=[
            pl.BlockSpec((NC, blk, D2), lambda i: (0, i, 0)),
            pl.BlockSpec((NC, blk, D2), lambda i: (0, i, 0)),
            pl.BlockSpec((NC, blk, DEGW), lambda i: (0, i, 0)),
            pl.BlockSpec((1, D1), lambda i: (0, 0)),
            pl.BlockSpec((D2, D1), lambda i: (0, 0)),
            pl.BlockSpec((D2, D1), lambda i: (0, 0)),
            pl.BlockSpec((D1, D2), lambda i: (0, 0)),
        ],
        out_specs=pl.BlockSpec((blk, D2), lambda i: (i, 0)),
        out_shape=jax.ShapeDtypeStruct((NPAD, D2), jnp.float32),
    )(pa, pb, degp, b1, w1a, w1b, w2)


def _out_layer(part, degp, b2, w, b3):
    din = part.shape[2]
    dout = w.shape[1]
    blk = N // 10
    return pl.pallas_call(
        _out_body,
        grid=(10,),
        in_specs=[
            pl.BlockSpec((NC, blk, din), lambda i: (0, i, 0)),
            pl.BlockSpec((NC, blk, DEGW), lambda i: (0, i, 0)),
            pl.BlockSpec((1, din), lambda i: (0, 0)),
            pl.BlockSpec((din, dout), lambda i: (0, 0)),
            pl.BlockSpec((1, dout), lambda i: (0, 0)),
        ],
        out_specs=pl.BlockSpec((blk, dout), lambda i: (i, 0)),
        out_shape=jax.ShapeDtypeStruct((N, dout), jnp.float32),
    )(part, degp, b2, w, b3)


def kernel(x, edge_index, n_nodes, W1, b1, W2, b2, W3, b3):
    row2d = edge_index[0].reshape(NT, NB, K)
    col2d = edge_index[1].reshape(NT, NB, K)
    zeros2 = jnp.zeros((RPT, D2), jnp.float32)
    zeros8 = jnp.zeros((RPT, DEGW), jnp.float32)
    ones8 = jnp.ones((K, DEGW), jnp.float32)
    xa = x[:, :D2]
    xb = x[:, D2:]

    degp = _deg(col2d, ones8, zeros8)
    pa = _agg64(xa, row2d, col2d, zeros2)
    pb = _agg64(xb, row2d, col2d, zeros2)
    y2 = _layer(pa, pb, degp, b1.reshape(1, D1),
                W1[:D2, :], W1[D2:, :], W2)
    part2 = _agg64(y2, row2d, col2d, zeros2)
    out = _out_layer(part2, degp, b2.reshape(1, D2), W3, b3.reshape(1, D2))
    return out


# speedup vs baseline: 11.4186x; 1.0694x over previous
import functools

import jax
import jax.numpy as jnp
from jax import lax
from jax.experimental import pallas as pl
from jax.experimental.pallas import tpu as pltpu
from jax.experimental.pallas import tpu_sc as plsc

N = 10000
NPAD = 10240
E = 320000
D1 = 128
D2 = 64
NC = 2
NS = 16
NT = NC * NS
K = 100
NB = 100
NBUF = 8
DEGW = 8
RPT = NPAD // NS

_sc_mesh = plsc.VectorSubcoreMesh(core_axis_name="c", subcore_axis_name="s")


def _make_agg(D):
    def body(y, row2d, col2d, zeros, out, ridx, cidx, acc, bufs,
             gsems, ssems):
        c = lax.axis_index("c")
        s = lax.axis_index("s")
        tid = c * NS + s

        pltpu.sync_copy(row2d.at[tid], ridx)
        pltpu.sync_copy(col2d.at[tid], cidx)
        base = s * RPT
        pltpu.sync_copy(zeros, acc.at[pl.ds(base, RPT)])
        plsc.subcore_barrier()

        def pstep(b, carry):
            pltpu.make_async_copy(
                y.at[ridx.at[b]], bufs.at[b], gsems.at[b]).start()
            return carry

        lax.fori_loop(0, NBUF, pstep, 0)

        def step(i, carry):
            @pl.when(i < NB)
            def _work():
                b = lax.rem(i, NBUF)
                pltpu.make_async_copy(y.at[ridx.at[i]], bufs.at[b],
                                      gsems.at[b]).wait()
                pltpu.make_async_copy(bufs.at[b], acc.at[cidx.at[i]],
                                      ssems.at[b]).start(add=True)

            @pl.when(i >= 1)
            def _retire():
                pi = i - 1
                pb = lax.rem(pi, NBUF)
                pltpu.make_async_copy(bufs.at[pb], acc.at[cidx.at[pi]],
                                      ssems.at[pb]).wait()

                @pl.when(pi + NBUF < NB)
                def _refill():
                    pltpu.make_async_copy(y.at[ridx.at[pi + NBUF]],
                                          bufs.at[pb], gsems.at[pb]).start()

            return carry

        lax.fori_loop(0, NB + 1, step, 0)

        plsc.subcore_barrier()
        pltpu.sync_copy(acc.at[pl.ds(base, RPT)],
                        out.at[c].at[pl.ds(base, RPT)])

    return pl.kernel(
        body,
        out_type=jax.ShapeDtypeStruct((NC, NPAD, D), jnp.float32),
        scratch_types=[
            pltpu.VMEM((NB, K), jnp.int32),
            pltpu.VMEM((NB, K), jnp.int32),
            pltpu.VMEM_SHARED((NPAD, D), jnp.float32),
            pltpu.VMEM((NBUF, K, D), jnp.float32),
            pltpu.SemaphoreType.DMA((NBUF,)),
            pltpu.SemaphoreType.DMA((NBUF,)),
        ],
        mesh=_sc_mesh,
        compiler_params=pltpu.CompilerParams(use_tc_tiling_on_sc=False),
    )


def _deg_body(col2d, ones, zeros8, degout, cidx, onesv, dacc, dsem):
    c = lax.axis_index("c")
    s = lax.axis_index("s")
    tid = c * NS + s
    pltpu.sync_copy(col2d.at[tid], cidx)
    pltpu.sync_copy(ones, onesv)
    base = s * RPT
    pltpu.sync_copy(zeros8, dacc.at[pl.ds(base, RPT)])
    plsc.subcore_barrier()

    def step(i, carry):
        pltpu.make_async_copy(onesv, dacc.at[cidx.at[i]],
                              dsem).start(add=True)

        @pl.when(i >= DEG_DEPTH)
        def _drain():
            pltpu.make_async_copy(onesv, dacc.at[cidx.at[i - DEG_DEPTH]],
                                  dsem).wait()

        return carry

    lax.fori_loop(0, NB, step, 0)

    def dstep(j, carry):
        pltpu.make_async_copy(onesv, dacc.at[cidx.at[j]], dsem).wait()
        return carry

    lax.fori_loop(0, DEG_DEPTH, dstep, 0)

    plsc.subcore_barrier()
    pltpu.sync_copy(dacc.at[pl.ds(base, RPT)],
                    degout.at[c].at[pl.ds(base, RPT)])


DEG_DEPTH = 8

_deg = pl.kernel(
    _deg_body,
    out_type=jax.ShapeDtypeStruct((NC, NPAD, DEGW), jnp.float32),
    scratch_types=[
        pltpu.VMEM((NB, K), jnp.int32),
        pltpu.VMEM((K, DEGW), jnp.float32),
        pltpu.VMEM_SHARED((NPAD, DEGW), jnp.float32),
        pltpu.SemaphoreType.DMA,
    ],
    mesh=_sc_mesh,
    compiler_params=pltpu.CompilerParams(use_tc_tiling_on_sc=False),
)


_agg64 = _make_agg(D2)


def _layer_body(pa_ref, pb_ref, d_ref, b_ref, w1a_ref, w1b_ref, w2_ref,
                o_ref):
    d = jnp.maximum(d_ref[0, :, 0:1] + d_ref[1, :, 0:1], 1.0)
    ha = (pa_ref[0] + pa_ref[1]) / d
    hb = (pb_ref[0] + pb_ref[1]) / d
    h = jnp.maximum(
        jnp.dot(ha, w1a_ref[...], preferred_element_type=jnp.float32)
        + jnp.dot(hb, w1b_ref[...], preferred_element_type=jnp.float32)
        + b_ref[...], 0.0)
    o_ref[...] = jnp.dot(h, w2_ref[...], preferred_element_type=jnp.float32)


def _out_body(p_ref, d_ref, b2_ref, w_ref, b3_ref, o_ref):
    p = p_ref[0] + p_ref[1]
    d = jnp.maximum(d_ref[0, :, 0:1] + d_ref[1, :, 0:1], 1.0)
    h = jnp.maximum(p / d + b2_ref[...], 0.0)
    o_ref[...] = (jnp.dot(h, w_ref[...], preferred_element_type=jnp.float32)
                  + b3_ref[...])


def _layer(pa, pb, degp, b1, w1a, w1b, w2):
    blk = NPAD // 10
    return pl.pallas_call(
        _layer_body,
        grid=(10,),
        in_specs=[
            pl.BlockSpec((NC, blk, D2), lambda i: (0, i, 0)),
            pl.BlockSpec((NC, blk, D2), lambda i: (0, i, 0)),
            pl.BlockSpec((NC, blk, DEGW), lambda i: (0, i, 0)),
            pl.BlockSpec((1, D1), lambda i: (0, 0)),
            pl.BlockSpec((D2, D1), lambda i: (0, 0)),
            pl.BlockSpec((D2, D1), lambda i: (0, 0)),
            pl.BlockSpec((D1, D2), lambda i: (0, 0)),
        ],
        out_specs=pl.BlockSpec((blk, D2), lambda i: (i, 0)),
        out_shape=jax.ShapeDtypeStruct((NPAD, D2), jnp.float32),
    )(pa, pb, degp, b1, w1a, w1b, w2)


def _out_layer(part, degp, b2, w, b3):
    din = part.shape[2]
    dout = w.shape[1]
    blk = N // 10
    return pl.pallas_call(
        _out_body,
        grid=(10,),
        in_specs=[
            pl.BlockSpec((NC, blk, din), lambda i: (0, i, 0)),
            pl.BlockSpec((NC, blk, DEGW), lambda i: (0, i, 0)),
            pl.BlockSpec((1, din), lambda i: (0, 0)),
            pl.BlockSpec((din, dout), lambda i: (0, 0)),
            pl.BlockSpec((1, dout), lambda i: (0, 0)),
        ],
        out_specs=pl.BlockSpec((blk, dout), lambda i: (i, 0)),
        out_shape=jax.ShapeDtypeStruct((N, dout), jnp.float32),
    )(part, degp, b2, w, b3)


def kernel(x, edge_index, n_nodes, W1, b1, W2, b2, W3, b3):
    row2d = edge_index[0].reshape(NT, NB, K)
    col2d = edge_index[1].reshape(NT, NB, K)
    zeros2 = jnp.zeros((RPT, D2), jnp.float32)
    zeros8 = jnp.zeros((RPT, DEGW), jnp.float32)
    ones8 = jnp.ones((K, DEGW), jnp.float32)
    xa = x[:, :D2]
    xb = x[:, D2:]

    degp = _deg(col2d, ones8, zeros8)
    pa = _agg64(xa, row2d, col2d, zeros2)
    pb = _agg64(xb, row2d, col2d, zeros2)
    y2 = _layer(pa, pb, degp, b1.reshape(1, D1),
                W1[:D2, :], W1[D2:, :], W2)
    part2 = _agg64(y2, row2d, col2d, zeros2)
    out = _out_layer(part2, degp, b2.reshape(1, D2), W3, b3.reshape(1, D2))
    return out
